# baseline (speedup 1.0000x reference)
import functools

import jax
import jax.numpy as jnp
import numpy as np
from jax import lax
from jax.experimental import pallas as pl
from jax.experimental.pallas import tpu as pltpu
from jax.experimental.pallas import tpu_sc as plsc

NS = 16384
NE = 2048
NO = 4096
ND = 512

SR = np.float32(0.1)
HALF = np.float32(0.5)
P3 = np.float32(0.3)
LO = np.float32(np.nextafter(np.float32(-1.0), np.float32(0.0)))
SQRT2 = np.float32(np.sqrt(2.0))
TINY = np.float32(np.finfo(np.float32).tiny)
BIG = np.float32(1e9)

M32 = 0xFFFFFFFF


def _tf_host(k, c):
    def rotl(x, d):
        return ((x << d) | (x >> (32 - d))) & M32
    ks = [k[0], k[1], 0x1BD11BDA ^ k[0] ^ k[1]]
    x0 = (c[0] + ks[0]) & M32
    x1 = (c[1] + ks[1]) & M32
    rots = [[13, 15, 26, 6], [17, 29, 16, 24]]
    for i in range(5):
        for r in rots[i % 2]:
            x0 = (x0 + x1) & M32
            x1 = rotl(x1, r)
            x1 = x0 ^ x1
        x0 = (x0 + ks[(i + 1) % 3]) & M32
        x1 = (x1 + ks[(i + 2) % 3] + i + 1) & M32
    return x0, x1


_KEY = (0, 42)
_K1 = _tf_host(_KEY, (0, 0))
_K2 = _tf_host(_KEY, (0, 1))
_K3 = _tf_host(_KEY, (0, 2))
_K4 = _tf_host(_KEY, (0, 3))
_KK = tuple(_tf_host(_K2, (0, t)) for t in range(3))


def _tf_bits(key, cnt):
    ks = [jnp.uint32(key[0]), jnp.uint32(key[1]),
          jnp.uint32(0x1BD11BDA ^ key[0] ^ key[1])]
    x0 = jnp.full(cnt.shape, ks[0], jnp.uint32)
    x1 = cnt + ks[1]
    rots = ((13, 15, 26, 6), (17, 29, 16, 24))
    for g in range(5):
        for r in rots[g % 2]:
            x0 = x0 + x1
            x1 = (x1 << jnp.uint32(r)) | (x1 >> jnp.uint32(32 - r))
            x1 = x0 ^ x1
        x0 = x0 + ks[(g + 1) % 3]
        x1 = x1 + ks[(g + 2) % 3] + jnp.uint32(g + 1)
    return x0 ^ x1


def _mantissa(bits):
    fb = (bits >> jnp.uint32(9)) | jnp.uint32(0x3F800000)
    return lax.bitcast_convert_type(fb, jnp.float32) - jnp.float32(1.0)


_ERF_A = [2.81022636e-08, 3.43273939e-07, -3.5233877e-06, -4.39150654e-06,
          0.00021858087, -0.00125372503, -0.00417768164, 0.246640727,
          1.50140941]
_ERF_B = [-0.000200214257, 0.000100950558, 0.00134934322, -0.00367342844,
          0.00573950773, -0.0076224613, 0.00943887047, 1.00167406,
          2.83297682]


def _erf_inv(x):
    w = -jnp.log1p(x * -x)
    wl = w - np.float32(2.5)
    wg = jnp.sqrt(w) - np.float32(3.0)
    pa = jnp.full_like(x, np.float32(_ERF_A[0]))
    pb = jnp.full_like(x, np.float32(_ERF_B[0]))
    for i in range(1, 9):
        pa = np.float32(_ERF_A[i]) + pa * wl
        pb = np.float32(_ERF_B[i]) + pb * wg
    return jnp.where(w < np.float32(5.0), pa, pb) * x


def _normal(key, cnt):
    m = _mantissa(_tf_bits(key, cnt))
    u = jnp.maximum(LO, m * jnp.float32(2.0) + LO)
    return SQRT2 * _erf_inv(u)


def _col_to_row(col):
    n = col.shape[0]
    sel = (lax.broadcasted_iota(jnp.int32, (n, n), 0)
           == lax.broadcasted_iota(jnp.int32, (n, n), 1))
    return jnp.sum(jnp.where(sel, col, jnp.zeros_like(col)), axis=0,
                   keepdims=True)


def _row_to_col(row):
    n = row.shape[1]
    sel = (lax.broadcasted_iota(jnp.int32, (n, n), 0)
           == lax.broadcasted_iota(jnp.int32, (n, n), 1))
    return jnp.sum(jnp.where(sel, row, jnp.zeros_like(row)), axis=1,
                   keepdims=True)


_RB = 1024


def _scout_body(scout_ref, src_ref, efit_ref, fit_s):
    b = pl.program_id(0)
    pos = scout_ref[...]
    rows = lax.broadcasted_iota(jnp.int32, (_RB, ND), 0) + b * _RB
    cols = lax.broadcasted_iota(jnp.int32, (_RB, ND), 1)
    cnt = (rows * ND + cols).astype(jnp.uint32)
    pos = pos + _normal(_K1, cnt) * SR
    fit = jnp.sqrt(jnp.sum(pos * pos, axis=1, keepdims=True))
    fit_s[pl.ds(b, 1), :] = _col_to_row(fit)

    @pl.when(b == NS // _RB - 1)
    def _():
        _select(fit_s[...], src_ref, efit_ref)


def _stage_a_sel(scout):
    return pl.pallas_call(
        _scout_body,
        grid=(NS // _RB,),
        in_specs=[pl.BlockSpec((_RB, ND), lambda b: (b, 0))],
        out_specs=[pl.BlockSpec((1, NE), lambda b: (0, 0)),
                   pl.BlockSpec((1, NE), lambda b: (0, 0))],
        out_shape=[jax.ShapeDtypeStruct((1, NE), jnp.int32),
                   jax.ShapeDtypeStruct((1, NE), jnp.float32)],
        scratch_shapes=[pltpu.VMEM((NS // _RB, _RB), jnp.float32)],
    )(scout)


_CH = 1024
_NCH = NS // _CH
_NSP = 2048
_NCAND = 2304


def _count_below(fit, s_col):
    c = jnp.zeros((_NSP, 1), jnp.float32)
    one = jnp.ones((_NSP, _CH), jnp.float32)
    zero = jnp.zeros((_NSP, _CH), jnp.float32)
    for k in range(_NCH):
        c = c + jnp.sum(jnp.where(fit[k:k + 1, :] < s_col, one, zero),
                        axis=1, keepdims=True)
    return c


def _select(fit, src_ref, efit_ref):
    ne_f = np.float32(NE)
    lo = jnp.min(fit)
    hi = jnp.max(fit) * np.float32(1.0 + 1e-5) + np.float32(1e-30)
    b_io = (lax.broadcasted_iota(jnp.int32, (_NSP, 1), 0).astype(jnp.float32)
            + np.float32(1.0))
    s1 = lo + b_io * ((hi - lo) / np.float32(_NSP))
    c1 = _count_below(fit, s1)
    lo2 = jnp.maximum(jnp.max(jnp.where(c1 < ne_f, s1, -BIG)), lo)
    hi2 = jnp.min(jnp.where(c1 >= ne_f, s1, BIG))
    s2 = lo2 + b_io * ((hi2 - lo2) / np.float32(_NSP))
    c2 = _count_below(fit, s2)
    t = jnp.minimum(jnp.min(jnp.where(c2 >= ne_f, s2, BIG)), hi2)
    tri = (lax.broadcasted_iota(jnp.int32, (_CH, _CH), 0)
           < lax.broadcasted_iota(jnp.int32, (_CH, _CH), 1))
    p_row = lax.broadcasted_iota(jnp.int32, (1, _NCAND), 1).astype(jnp.float32)
    comp_fit = jnp.zeros((1, _NCAND), jnp.float32)
    comp_idx = jnp.zeros((1, _NCAND), jnp.float32)
    matched = jnp.zeros((1, _NCAND), jnp.float32)
    offs = jnp.float32(0.0)
    zc = jnp.zeros((_CH, _CH), jnp.float32)
    zn = jnp.zeros((_CH, _NCAND), jnp.float32)
    for k in range(_NCH):
        frow = fit[k:k + 1, :]
        fcol = _row_to_col(frow)
        cand_col = fcol < t
        w = jnp.sum(jnp.where(cand_col & tri, jnp.ones_like(zc), zc),
                    axis=0, keepdims=True)
        pc_col = _row_to_col(w) + offs
        offs = offs + jnp.sum(jnp.where(cand_col, jnp.ones((_CH, 1),
                                                           jnp.float32),
                                        jnp.zeros((_CH, 1), jnp.float32)))
        m = cand_col & (pc_col == p_row)
        comp_fit = comp_fit + jnp.sum(jnp.where(m, fcol, zn), axis=0,
                                      keepdims=True)
        i_col = (lax.broadcasted_iota(jnp.int32, (_CH, 1), 0)
                 + k * _CH).astype(jnp.float32)
        comp_idx = comp_idx + jnp.sum(jnp.where(m, i_col, zn), axis=0,
                                      keepdims=True)
        matched = matched + jnp.sum(jnp.where(m, jnp.ones_like(zn), zn),
                                    axis=0, keepdims=True)
    comp_fit = comp_fit + (np.float32(1.0) - matched) * BIG
    mine_col = _row_to_col(comp_fit)
    tri_c = (lax.broadcasted_iota(jnp.int32, (_NCAND, _NCAND), 0)
             < lax.broadcasted_iota(jnp.int32, (_NCAND, _NCAND), 1))
    less = comp_fit < mine_col
    tie = (comp_fit == mine_col) & tri_c
    rank_col = jnp.sum((less | tie).astype(jnp.float32), axis=1,
                       keepdims=True)
    slot_row = lax.broadcasted_iota(jnp.int32, (1, NE), 1).astype(jnp.float32)
    idx_col = _row_to_col(comp_idx)
    fit_col = mine_col
    eq = rank_col == slot_row
    ze = jnp.zeros((_NCAND, NE), jnp.float32)
    src_ref[...] = jnp.sum(jnp.where(eq, idx_col, ze), axis=0,
                           keepdims=True).astype(jnp.int32)
    efit_ref[...] = jnp.sum(jnp.where(eq, fit_col, ze), axis=0, keepdims=True)


def _sc_gather_rows(table, idx):
    b_tot = idx.shape[0]
    nw = 32
    bpw = b_tot // nw
    mesh = plsc.VectorSubcoreMesh(core_axis_name="c", subcore_axis_name="s")

    @functools.partial(
        pl.kernel, mesh=mesh,
        out_type=jax.ShapeDtypeStruct((b_tot, ND), jnp.float32),
        scratch_types=[pltpu.VMEM((bpw,), jnp.int32),
                       pltpu.VMEM((bpw, ND), jnp.float32),
                       pltpu.SemaphoreType.DMA],
    )
    def k(table_hbm, idx_hbm, out_hbm, idx_v, rows_v, sem):
        wid = lax.axis_index("s") * 2 + lax.axis_index("c")
        base = wid * bpw
        pltpu.sync_copy(idx_hbm.at[pl.ds(base, bpw)], idx_v)
        pltpu.async_copy(table_hbm.at[idx_v], rows_v, sem).wait()
        pltpu.sync_copy(rows_v, out_hbm.at[pl.ds(base, bpw)])

    return k(table, idx)


_EB = 256


def _elite_body(ebase_ref, src_ref, efit_ref, epos_ref, efit2_ref):
    e = pl.program_id(0)
    src = _row_to_col(src_ref[...])
    fit = _row_to_col(efit_ref[...])
    d_io = lax.broadcasted_iota(jnp.int32, (_EB, ND), 1)
    cnt = (src * ND + d_io).astype(jnp.uint32)
    pos = ebase_ref[...] + _normal(_K1, cnt) * SR
    slot = e * _EB + lax.broadcasted_iota(jnp.int32, (_EB, ND), 0)
    cnt_e = (slot * ND + d_io).astype(jnp.uint32)
    for t in range(3):
        local = pos + (_normal(_KK[t], cnt_e) * SR) * HALF
        lfit = jnp.sqrt(jnp.sum(local * local, axis=1, keepdims=True))
        imp = lfit < fit
        pos = jnp.where(imp, local, pos)
        fit = jnp.where(imp, lfit, fit)
    epos_ref[...] = pos
    efit2_ref[...] = _col_to_row(fit)


def _stage_e(ebase, src, efit):
    return pl.pallas_call(
        _elite_body,
        grid=(NE // _EB,),
        in_specs=[pl.BlockSpec((_EB, ND), lambda e: (e, 0)),
                  pl.BlockSpec((1, _EB), lambda e: (0, e)),
                  pl.BlockSpec((1, _EB), lambda e: (0, e))],
        out_specs=[pl.BlockSpec((_EB, ND), lambda e: (e, 0)),
                   pl.BlockSpec((1, _EB), lambda e: (0, e))],
        out_shape=[jax.ShapeDtypeStruct((NE, ND), jnp.float32),
                   jax.ShapeDtypeStruct((1, NE), jnp.float32)],
    )(ebase, src, efit)


_FB = 512


def _cat_body(efit2_ref, eidx_ref):
    b = pl.program_id(0)
    logits = -efit2_ref[...]
    s_io = b * _FB + lax.broadcasted_iota(jnp.int32, (_FB, NE), 0)
    e_io = lax.broadcasted_iota(jnp.int32, (_FB, NE), 1)
    cnt = (s_io * NE + e_io).astype(jnp.uint32)
    u = jnp.maximum(_mantissa(_tf_bits(_K3, cnt)), TINY)
    val = -jnp.log(-jnp.log(u)) + logits
    mx = jnp.max(val, axis=1, keepdims=True)
    idx = jnp.min(jnp.where(val == mx, e_io.astype(jnp.float32), BIG),
                  axis=1, keepdims=True)
    eidx_ref[...] = _col_to_row(idx).astype(jnp.int32).reshape(1, 1, _FB)


def _stage_f(efit2):
    return pl.pallas_call(
        _cat_body,
        grid=(NO // _FB,),
        in_specs=[pl.BlockSpec((1, NE), lambda b: (0, 0))],
        out_specs=pl.BlockSpec((1, 1, _FB), lambda b: (b, 0, 0)),
        out_shape=jax.ShapeDtypeStruct((NO // _FB, 1, _FB), jnp.int32),
    )(efit2)


_OB = 512


def _onlooker_body(obase_ref, efit2_ref, efinal_ref, bp_ref, bf_ref, out_ref,
                   bfit_s, brow_s):
    b = pl.program_id(0)
    obase = obase_ref[...]
    rows = b * _OB + lax.broadcasted_iota(jnp.int32, (_OB, ND), 0)
    cols = lax.broadcasted_iota(jnp.int32, (_OB, ND), 1)
    cnt = (rows * ND + cols).astype(jnp.uint32)
    local = obase + (_normal(_K4, cnt) * SR) * P3
    lfit = jnp.sqrt(jnp.sum(local * local, axis=1, keepdims=True))
    cfit = jnp.sqrt(jnp.sum(obase * obase, axis=1, keepdims=True))
    pos = jnp.where(lfit < cfit, local, obase)
    bm = jnp.min(lfit)
    r_io = lax.broadcasted_iota(jnp.int32, (_OB, 1), 0).astype(jnp.float32)
    rstar = jnp.min(jnp.where(lfit == bm, r_io, BIG))
    mask = r_io == rstar
    cand = jnp.sum(jnp.where(mask, pos, jnp.zeros_like(pos)),
                   axis=0, keepdims=True)
    bm_v = jnp.zeros((1, 128), jnp.float32) + bm

    @pl.when(b == 0)
    def _():
        bfit_s[...] = jnp.full((1, 128), BIG, jnp.float32)

    prev = bfit_s[...]
    upd = bm_v[:, 0:1] < prev[:, 0:1]
    bfit_s[...] = jnp.where(upd, bm_v, prev)
    brow_s[...] = jnp.where(upd, cand, brow_s[...])

    @pl.when(b == NO // _OB - 1)
    def _():
        efit2 = efit2_ref[...]
        em = jnp.min(efit2)
        e_io = lax.broadcasted_iota(jnp.int32, (1, NE), 1).astype(jnp.float32)
        estar = jnp.min(jnp.where(efit2 == em, e_io, BIG))
        r_io2 = lax.broadcasted_iota(jnp.int32, (NE, 1), 0).astype(jnp.float32)
        efinal = efinal_ref[...]
        erow = jnp.sum(jnp.where(r_io2 == estar, efinal,
                                 jnp.zeros_like(efinal)),
                       axis=0, keepdims=True)
        om = bfit_s[...][:, 0:1]
        em_v = jnp.zeros((1, 1), jnp.float32) + em
        ewin = em_v <= om
        brow = jnp.where(ewin, erow, brow_s[...])
        bfit = jnp.minimum(em_v, om)
        better = bfit < bf_ref[0:1, 0:1]
        outrow = jnp.where(better, brow, bp_ref[...])
        out_ref[...] = jnp.broadcast_to(outrow, out_ref.shape)


def _stage_hi(obase, efit2, efinal, bp, bf, batch):
    return pl.pallas_call(
        _onlooker_body,
        grid=(NO // _OB,),
        in_specs=[pl.BlockSpec((_OB, ND), lambda b: (b, 0)),
                  pl.BlockSpec((1, NE), lambda b: (0, 0)),
                  pl.BlockSpec((NE, ND), lambda b: (0, 0)),
                  pl.BlockSpec((1, ND), lambda b: (0, 0)),
                  pl.BlockSpec((1, 128), lambda b: (0, 0))],
        out_specs=pl.BlockSpec((batch, ND), lambda b: (0, 0)),
        out_shape=jax.ShapeDtypeStruct((batch, ND), jnp.float32),
        scratch_shapes=[pltpu.VMEM((1, 128), jnp.float32),
                        pltpu.VMEM((1, ND), jnp.float32)],
    )(obase, efit2, efinal, bp, bf)


def kernel(x, scout_positions, elite_positions, onlooker_positions,
           best_position, best_fitness):
    batch = x.shape[0]
    src, efit = _stage_a_sel(scout_positions)
    ebase = _sc_gather_rows(scout_positions, src.reshape(NE))
    efinal, efit2 = _stage_e(ebase, src, efit)
    eidx = _stage_f(efit2)
    obase = _sc_gather_rows(efinal, eidx.reshape(NO))
    bp = best_position.reshape(1, ND)
    bf = jnp.broadcast_to(best_fitness.reshape(1, 1), (1, 128))
    return _stage_hi(obase, efit2, efinal, bp, bf, batch)


# speedup vs baseline: 1.5165x; 1.5165x over previous
import functools

import jax
import jax.numpy as jnp
import numpy as np
from jax import lax
from jax.experimental import pallas as pl
from jax.experimental.pallas import tpu as pltpu
from jax.experimental.pallas import tpu_sc as plsc

NS = 16384
NE = 2048
NO = 4096
ND = 512

SR = np.float32(0.1)
HALF = np.float32(0.5)
P3 = np.float32(0.3)
LO = np.float32(np.nextafter(np.float32(-1.0), np.float32(0.0)))
SQRT2 = np.float32(np.sqrt(2.0))
TINY = np.float32(np.finfo(np.float32).tiny)
BIG = np.float32(1e9)

M32 = 0xFFFFFFFF


def _tf_host(k, c):
    def rotl(x, d):
        return ((x << d) | (x >> (32 - d))) & M32
    ks = [k[0], k[1], 0x1BD11BDA ^ k[0] ^ k[1]]
    x0 = (c[0] + ks[0]) & M32
    x1 = (c[1] + ks[1]) & M32
    rots = [[13, 15, 26, 6], [17, 29, 16, 24]]
    for i in range(5):
        for r in rots[i % 2]:
            x0 = (x0 + x1) & M32
            x1 = rotl(x1, r)
            x1 = x0 ^ x1
        x0 = (x0 + ks[(i + 1) % 3]) & M32
        x1 = (x1 + ks[(i + 2) % 3] + i + 1) & M32
    return x0, x1


_KEY = (0, 42)
_K1 = _tf_host(_KEY, (0, 0))
_K2 = _tf_host(_KEY, (0, 1))
_K3 = _tf_host(_KEY, (0, 2))
_K4 = _tf_host(_KEY, (0, 3))
_KK = tuple(_tf_host(_K2, (0, t)) for t in range(3))


def _tf_bits(key, cnt):
    ks = [jnp.uint32(key[0]), jnp.uint32(key[1]),
          jnp.uint32(0x1BD11BDA ^ key[0] ^ key[1])]
    x0 = jnp.full(cnt.shape, ks[0], jnp.uint32)
    x1 = cnt + ks[1]
    rots = ((13, 15, 26, 6), (17, 29, 16, 24))
    for g in range(5):
        for r in rots[g % 2]:
            x0 = x0 + x1
            x1 = (x1 << jnp.uint32(r)) | (x1 >> jnp.uint32(32 - r))
            x1 = x0 ^ x1
        x0 = x0 + ks[(g + 1) % 3]
        x1 = x1 + ks[(g + 2) % 3] + jnp.uint32(g + 1)
    return x0 ^ x1


def _mantissa(bits):
    fb = (bits >> jnp.uint32(9)) | jnp.uint32(0x3F800000)
    return lax.bitcast_convert_type(fb, jnp.float32) - jnp.float32(1.0)


_ERF_A = [2.81022636e-08, 3.43273939e-07, -3.5233877e-06, -4.39150654e-06,
          0.00021858087, -0.00125372503, -0.00417768164, 0.246640727,
          1.50140941]
_ERF_B = [-0.000200214257, 0.000100950558, 0.00134934322, -0.00367342844,
          0.00573950773, -0.0076224613, 0.00943887047, 1.00167406,
          2.83297682]


def _erf_inv(x):
    w = -jnp.log1p(x * -x)
    wl = w - np.float32(2.5)
    wg = jnp.sqrt(w) - np.float32(3.0)
    pa = jnp.full_like(x, np.float32(_ERF_A[0]))
    pb = jnp.full_like(x, np.float32(_ERF_B[0]))
    for i in range(1, 9):
        pa = np.float32(_ERF_A[i]) + pa * wl
        pb = np.float32(_ERF_B[i]) + pb * wg
    return jnp.where(w < np.float32(5.0), pa, pb) * x


def _normal(key, cnt):
    m = _mantissa(_tf_bits(key, cnt))
    u = jnp.maximum(LO, m * jnp.float32(2.0) + LO)
    return SQRT2 * _erf_inv(u)


def _col_to_row(col):
    n = col.shape[0]
    sel = (lax.broadcasted_iota(jnp.int32, (n, n), 0)
           == lax.broadcasted_iota(jnp.int32, (n, n), 1))
    return jnp.sum(jnp.where(sel, col, jnp.zeros_like(col)), axis=0,
                   keepdims=True)


def _row_to_col(row):
    n = row.shape[1]
    sel = (lax.broadcasted_iota(jnp.int32, (n, n), 0)
           == lax.broadcasted_iota(jnp.int32, (n, n), 1))
    return jnp.sum(jnp.where(sel, row, jnp.zeros_like(row)), axis=1,
                   keepdims=True)


_RB = 1024


def _scout_body(scout_ref, fit_ref):
    b = pl.program_id(0)
    pos = scout_ref[...]
    rows = lax.broadcasted_iota(jnp.int32, (_RB, ND), 0) + b * _RB
    cols = lax.broadcasted_iota(jnp.int32, (_RB, ND), 1)
    cnt = (rows * ND + cols).astype(jnp.uint32)
    pos = pos + _normal(_K1, cnt) * SR
    fit = jnp.sqrt(jnp.sum(pos * pos, axis=1, keepdims=True))
    fit_ref[...] = _col_to_row(fit).reshape(1, 1, _RB)


def _stage_a(scout):
    return pl.pallas_call(
        _scout_body,
        grid=(NS // _RB,),
        in_specs=[pl.BlockSpec((_RB, ND), lambda b: (b, 0))],
        out_specs=pl.BlockSpec((1, 1, _RB), lambda b: (b, 0, 0)),
        out_shape=jax.ShapeDtypeStruct((NS // _RB, 1, _RB), jnp.float32),
    )(scout)


_CH = 1024
_NCH = NS // _CH
_NSP = 2048
_NCAND = 2304


def _count_below(fit, s_col):
    c = jnp.zeros((_NSP, 1), jnp.float32)
    one = jnp.ones((_NSP, _CH), jnp.float32)
    zero = jnp.zeros((_NSP, _CH), jnp.float32)
    for k in range(_NCH):
        c = c + jnp.sum(jnp.where(fit[k:k + 1, :] < s_col, one, zero),
                        axis=1, keepdims=True)
    return c


def _select_body(fit_ref, src_ref, efit_ref):
    fit = fit_ref[...].reshape(_NCH, _CH)
    ne_f = np.float32(NE)
    lo = jnp.min(fit)
    hi = jnp.max(fit) * np.float32(1.0 + 1e-5) + np.float32(1e-30)
    b_io = (lax.broadcasted_iota(jnp.int32, (_NSP, 1), 0).astype(jnp.float32)
            + np.float32(1.0))
    s1 = lo + b_io * ((hi - lo) / np.float32(_NSP))
    c1 = _count_below(fit, s1)
    lo2 = jnp.maximum(jnp.max(jnp.where(c1 < ne_f, s1, -BIG)), lo)
    hi2 = jnp.min(jnp.where(c1 >= ne_f, s1, BIG))
    s2 = lo2 + b_io * ((hi2 - lo2) / np.float32(_NSP))
    c2 = _count_below(fit, s2)
    t = jnp.minimum(jnp.min(jnp.where(c2 >= ne_f, s2, BIG)), hi2)
    tri = (lax.broadcasted_iota(jnp.int32, (_CH, _CH), 0)
           < lax.broadcasted_iota(jnp.int32, (_CH, _CH), 1))
    p_row = lax.broadcasted_iota(jnp.int32, (1, _NCAND), 1).astype(jnp.float32)
    comp_fit = jnp.zeros((1, _NCAND), jnp.float32)
    comp_idx = jnp.zeros((1, _NCAND), jnp.float32)
    matched = jnp.zeros((1, _NCAND), jnp.float32)
    offs = jnp.float32(0.0)
    zc = jnp.zeros((_CH, _CH), jnp.float32)
    zn = jnp.zeros((_CH, _NCAND), jnp.float32)
    for k in range(_NCH):
        frow = fit[k:k + 1, :]
        fcol = _row_to_col(frow)
        cand_col = fcol < t
        w = jnp.sum(jnp.where(cand_col & tri, jnp.ones_like(zc), zc),
                    axis=0, keepdims=True)
        pc_col = _row_to_col(w) + offs
        offs = offs + jnp.sum(jnp.where(cand_col, jnp.ones((_CH, 1),
                                                           jnp.float32),
                                        jnp.zeros((_CH, 1), jnp.float32)))
        m = cand_col & (pc_col == p_row)
        comp_fit = comp_fit + jnp.sum(jnp.where(m, fcol, zn), axis=0,
                                      keepdims=True)
        i_col = (lax.broadcasted_iota(jnp.int32, (_CH, 1), 0)
                 + k * _CH).astype(jnp.float32)
        comp_idx = comp_idx + jnp.sum(jnp.where(m, i_col, zn), axis=0,
                                      keepdims=True)
        matched = matched + jnp.sum(jnp.where(m, jnp.ones_like(zn), zn),
                                    axis=0, keepdims=True)
    comp_fit = comp_fit + (np.float32(1.0) - matched) * BIG
    mine_col = _row_to_col(comp_fit)
    tri_c = (lax.broadcasted_iota(jnp.int32, (_NCAND, _NCAND), 0)
             < lax.broadcasted_iota(jnp.int32, (_NCAND, _NCAND), 1))
    less = comp_fit < mine_col
    tie = (comp_fit == mine_col) & tri_c
    rank_col = jnp.sum((less | tie).astype(jnp.float32), axis=1,
                       keepdims=True)
    slot_row = lax.broadcasted_iota(jnp.int32, (1, NE), 1).astype(jnp.float32)
    idx_col = _row_to_col(comp_idx)
    fit_col = mine_col
    eq = rank_col == slot_row
    ze = jnp.zeros((_NCAND, NE), jnp.float32)
    src_ref[...] = jnp.sum(jnp.where(eq, idx_col, ze), axis=0,
                           keepdims=True).astype(jnp.int32)
    efit_ref[...] = jnp.sum(jnp.where(eq, fit_col, ze), axis=0, keepdims=True)


def _stage_bc(fit):
    return pl.pallas_call(
        _select_body,
        in_specs=[pl.BlockSpec((_NCH, 1, _CH), lambda: (0, 0, 0))],
        out_specs=[pl.BlockSpec((1, NE), lambda: (0, 0)),
                   pl.BlockSpec((1, NE), lambda: (0, 0))],
        out_shape=[jax.ShapeDtypeStruct((1, NE), jnp.int32),
                   jax.ShapeDtypeStruct((1, NE), jnp.float32)],
    )(fit)


def _sc_gather_rows(table, idx):
    b_tot = idx.shape[0]
    nw = 32
    bpw = b_tot // nw
    mesh = plsc.VectorSubcoreMesh(core_axis_name="c", subcore_axis_name="s")

    @functools.partial(
        pl.kernel, mesh=mesh,
        out_type=jax.ShapeDtypeStruct((b_tot, ND), jnp.float32),
        scratch_types=[pltpu.VMEM((bpw,), jnp.int32),
                       pltpu.VMEM((bpw, ND), jnp.float32),
                       pltpu.SemaphoreType.DMA],
    )
    def k(table_hbm, idx_hbm, out_hbm, idx_v, rows_v, sem):
        wid = lax.axis_index("s") * 2 + lax.axis_index("c")
        base = wid * bpw
        pltpu.sync_copy(idx_hbm.at[pl.ds(base, bpw)], idx_v)
        pltpu.async_copy(table_hbm.at[idx_v], rows_v, sem).wait()
        pltpu.sync_copy(rows_v, out_hbm.at[pl.ds(base, bpw)])

    return k(table, idx)


_EB = 256


def _elite_body(ebase_ref, src_ref, efit_ref, epos_ref, efit2_ref):
    e = pl.program_id(0)
    src = _row_to_col(src_ref[...])
    fit = _row_to_col(efit_ref[...])
    d_io = lax.broadcasted_iota(jnp.int32, (_EB, ND), 1)
    cnt = (src * ND + d_io).astype(jnp.uint32)
    pos = ebase_ref[...] + _normal(_K1, cnt) * SR
    slot = e * _EB + lax.broadcasted_iota(jnp.int32, (_EB, ND), 0)
    cnt_e = (slot * ND + d_io).astype(jnp.uint32)
    for t in range(3):
        local = pos + (_normal(_KK[t], cnt_e) * SR) * HALF
        lfit = jnp.sqrt(jnp.sum(local * local, axis=1, keepdims=True))
        imp = lfit < fit
        pos = jnp.where(imp, local, pos)
        fit = jnp.where(imp, lfit, fit)
    epos_ref[...] = pos
    efit2_ref[...] = _col_to_row(fit)


def _stage_e(ebase, src, efit):
    return pl.pallas_call(
        _elite_body,
        grid=(NE // _EB,),
        in_specs=[pl.BlockSpec((_EB, ND), lambda e: (e, 0)),
                  pl.BlockSpec((1, _EB), lambda e: (0, e)),
                  pl.BlockSpec((1, _EB), lambda e: (0, e))],
        out_specs=[pl.BlockSpec((_EB, ND), lambda e: (e, 0)),
                   pl.BlockSpec((1, _EB), lambda e: (0, e))],
        out_shape=[jax.ShapeDtypeStruct((NE, ND), jnp.float32),
                   jax.ShapeDtypeStruct((1, NE), jnp.float32)],
    )(ebase, src, efit)


_FB = 512


def _cat_body(efit2_ref, eidx_ref):
    b = pl.program_id(0)
    logits = -efit2_ref[...]
    s_io = b * _FB + lax.broadcasted_iota(jnp.int32, (_FB, NE), 0)
    e_io = lax.broadcasted_iota(jnp.int32, (_FB, NE), 1)
    cnt = (s_io * NE + e_io).astype(jnp.uint32)
    u = jnp.maximum(_mantissa(_tf_bits(_K3, cnt)), TINY)
    val = -jnp.log(-jnp.log(u)) + logits
    mx = jnp.max(val, axis=1, keepdims=True)
    idx = jnp.min(jnp.where(val == mx, e_io.astype(jnp.float32), BIG),
                  axis=1, keepdims=True)
    eidx_ref[...] = _col_to_row(idx).astype(jnp.int32).reshape(1, 1, _FB)


def _stage_f(efit2):
    return pl.pallas_call(
        _cat_body,
        grid=(NO // _FB,),
        in_specs=[pl.BlockSpec((1, NE), lambda b: (0, 0))],
        out_specs=pl.BlockSpec((1, 1, _FB), lambda b: (b, 0, 0)),
        out_shape=jax.ShapeDtypeStruct((NO // _FB, 1, _FB), jnp.int32),
    )(efit2)


_OB = 512


def _onlooker_body(obase_ref, efit2_ref, efinal_ref, bp_ref, bf_ref, out_ref,
                   bfit_s, brow_s):
    b = pl.program_id(0)
    obase = obase_ref[...]
    rows = b * _OB + lax.broadcasted_iota(jnp.int32, (_OB, ND), 0)
    cols = lax.broadcasted_iota(jnp.int32, (_OB, ND), 1)
    cnt = (rows * ND + cols).astype(jnp.uint32)
    local = obase + (_normal(_K4, cnt) * SR) * P3
    lfit = jnp.sqrt(jnp.sum(local * local, axis=1, keepdims=True))
    cfit = jnp.sqrt(jnp.sum(obase * obase, axis=1, keepdims=True))
    pos = jnp.where(lfit < cfit, local, obase)
    bm = jnp.min(lfit)
    r_io = lax.broadcasted_iota(jnp.int32, (_OB, 1), 0).astype(jnp.float32)
    rstar = jnp.min(jnp.where(lfit == bm, r_io, BIG))
    mask = r_io == rstar
    cand = jnp.sum(jnp.where(mask, pos, jnp.zeros_like(pos)),
                   axis=0, keepdims=True)
    bm_v = jnp.zeros((1, 128), jnp.float32) + bm

    @pl.when(b == 0)
    def _():
        bfit_s[...] = jnp.full((1, 128), BIG, jnp.float32)

    prev = bfit_s[...]
    upd = bm_v[:, 0:1] < prev[:, 0:1]
    bfit_s[...] = jnp.where(upd, bm_v, prev)
    brow_s[...] = jnp.where(upd, cand, brow_s[...])

    @pl.when(b == NO // _OB - 1)
    def _():
        efit2 = efit2_ref[...]
        em = jnp.min(efit2)
        e_io = lax.broadcasted_iota(jnp.int32, (1, NE), 1).astype(jnp.float32)
        estar = jnp.min(jnp.where(efit2 == em, e_io, BIG))
        r_io2 = lax.broadcasted_iota(jnp.int32, (NE, 1), 0).astype(jnp.float32)
        efinal = efinal_ref[...]
        erow = jnp.sum(jnp.where(r_io2 == estar, efinal,
                                 jnp.zeros_like(efinal)),
                       axis=0, keepdims=True)
        om = bfit_s[...][:, 0:1]
        em_v = jnp.zeros((1, 1), jnp.float32) + em
        ewin = em_v <= om
        brow = jnp.where(ewin, erow, brow_s[...])
        bfit = jnp.minimum(em_v, om)
        better = bfit < bf_ref[0:1, 0:1]
        outrow = jnp.where(better, brow, bp_ref[...])
        out_ref[...] = jnp.broadcast_to(outrow, out_ref.shape)


def _stage_hi(obase, efit2, efinal, bp, bf, batch):
    return pl.pallas_call(
        _onlooker_body,
        grid=(NO // _OB,),
        in_specs=[pl.BlockSpec((_OB, ND), lambda b: (b, 0)),
                  pl.BlockSpec((1, NE), lambda b: (0, 0)),
                  pl.BlockSpec((NE, ND), lambda b: (0, 0)),
                  pl.BlockSpec((1, ND), lambda b: (0, 0)),
                  pl.BlockSpec((1, 128), lambda b: (0, 0))],
        out_specs=pl.BlockSpec((batch, ND), lambda b: (0, 0)),
        out_shape=jax.ShapeDtypeStruct((batch, ND), jnp.float32),
        scratch_shapes=[pltpu.VMEM((1, 128), jnp.float32),
                        pltpu.VMEM((1, ND), jnp.float32)],
    )(obase, efit2, efinal, bp, bf)


def kernel(x, scout_positions, elite_positions, onlooker_positions,
           best_position, best_fitness):
    batch = x.shape[0]
    fit = _stage_a(scout_positions)
    src, efit = _stage_bc(fit)
    ebase = _sc_gather_rows(scout_positions, src.reshape(NE))
    efinal, efit2 = _stage_e(ebase, src, efit)
    eidx = _stage_f(efit2)
    obase = _sc_gather_rows(efinal, eidx.reshape(NO))
    bp = best_position.reshape(1, ND)
    bf = jnp.broadcast_to(best_fitness.reshape(1, 1), (1, 128))
    return _stage_hi(obase, efit2, efinal, bp, bf, batch)


# speedup vs baseline: 1.5730x; 1.0373x over previous
import functools

import jax
import jax.numpy as jnp
import numpy as np
from jax import lax
from jax.experimental import pallas as pl
from jax.experimental.pallas import tpu as pltpu
from jax.experimental.pallas import tpu_sc as plsc

NS = 16384
NE = 2048
NO = 4096
ND = 512

SR = np.float32(0.1)
HALF = np.float32(0.5)
P3 = np.float32(0.3)
LO = np.float32(np.nextafter(np.float32(-1.0), np.float32(0.0)))
SQRT2 = np.float32(np.sqrt(2.0))
TINY = np.float32(np.finfo(np.float32).tiny)
BIG = np.float32(1e9)

M32 = 0xFFFFFFFF


def _tf_host(k, c):
    def rotl(x, d):
        return ((x << d) | (x >> (32 - d))) & M32
    ks = [k[0], k[1], 0x1BD11BDA ^ k[0] ^ k[1]]
    x0 = (c[0] + ks[0]) & M32
    x1 = (c[1] + ks[1]) & M32
    rots = [[13, 15, 26, 6], [17, 29, 16, 24]]
    for i in range(5):
        for r in rots[i % 2]:
            x0 = (x0 + x1) & M32
            x1 = rotl(x1, r)
            x1 = x0 ^ x1
        x0 = (x0 + ks[(i + 1) % 3]) & M32
        x1 = (x1 + ks[(i + 2) % 3] + i + 1) & M32
    return x0, x1


_KEY = (0, 42)
_K1 = _tf_host(_KEY, (0, 0))
_K2 = _tf_host(_KEY, (0, 1))
_K3 = _tf_host(_KEY, (0, 2))
_K4 = _tf_host(_KEY, (0, 3))
_KK = tuple(_tf_host(_K2, (0, t)) for t in range(3))


def _tf_bits(key, cnt):
    ks = [jnp.uint32(key[0]), jnp.uint32(key[1]),
          jnp.uint32(0x1BD11BDA ^ key[0] ^ key[1])]
    x0 = jnp.full(cnt.shape, ks[0], jnp.uint32)
    x1 = cnt + ks[1]
    rots = ((13, 15, 26, 6), (17, 29, 16, 24))
    for g in range(5):
        for r in rots[g % 2]:
            x0 = x0 + x1
            x1 = (x1 << jnp.uint32(r)) | (x1 >> jnp.uint32(32 - r))
            x1 = x0 ^ x1
        x0 = x0 + ks[(g + 1) % 3]
        x1 = x1 + ks[(g + 2) % 3] + jnp.uint32(g + 1)
    return x0 ^ x1


def _mantissa(bits):
    fb = (bits >> jnp.uint32(9)) | jnp.uint32(0x3F800000)
    return lax.bitcast_convert_type(fb, jnp.float32) - jnp.float32(1.0)


_ERF_A = [2.81022636e-08, 3.43273939e-07, -3.5233877e-06, -4.39150654e-06,
          0.00021858087, -0.00125372503, -0.00417768164, 0.246640727,
          1.50140941]
_ERF_B = [-0.000200214257, 0.000100950558, 0.00134934322, -0.00367342844,
          0.00573950773, -0.0076224613, 0.00943887047, 1.00167406,
          2.83297682]


def _erf_inv(x):
    w = -jnp.log1p(x * -x)
    wl = w - np.float32(2.5)
    wg = jnp.sqrt(w) - np.float32(3.0)
    pa = jnp.full_like(x, np.float32(_ERF_A[0]))
    pb = jnp.full_like(x, np.float32(_ERF_B[0]))
    for i in range(1, 9):
        pa = np.float32(_ERF_A[i]) + pa * wl
        pb = np.float32(_ERF_B[i]) + pb * wg
    return jnp.where(w < np.float32(5.0), pa, pb) * x


def _normal(key, cnt):
    m = _mantissa(_tf_bits(key, cnt))
    u = jnp.maximum(LO, m * jnp.float32(2.0) + LO)
    return SQRT2 * _erf_inv(u)


def _col_to_row(col):
    n = col.shape[0]
    sel = (lax.broadcasted_iota(jnp.int32, (n, n), 0)
           == lax.broadcasted_iota(jnp.int32, (n, n), 1))
    return jnp.sum(jnp.where(sel, col, jnp.zeros_like(col)), axis=0,
                   keepdims=True)


def _row_to_col(row):
    n = row.shape[1]
    sel = (lax.broadcasted_iota(jnp.int32, (n, n), 0)
           == lax.broadcasted_iota(jnp.int32, (n, n), 1))
    return jnp.sum(jnp.where(sel, row, jnp.zeros_like(row)), axis=1,
                   keepdims=True)


_RB = 1024


def _scout_body(scout_ref, fit_ref, pos_ref):
    b = pl.program_id(0)
    pos = scout_ref[...]
    rows = lax.broadcasted_iota(jnp.int32, (_RB, ND), 0) + b * _RB
    cols = lax.broadcasted_iota(jnp.int32, (_RB, ND), 1)
    cnt = (rows * ND + cols).astype(jnp.uint32)
    pos = pos + _normal(_K1, cnt) * SR
    pos_ref[...] = pos
    fit = jnp.sqrt(jnp.sum(pos * pos, axis=1, keepdims=True))
    fit_ref[...] = _col_to_row(fit).reshape(1, 1, _RB)


def _stage_a(scout):
    return pl.pallas_call(
        _scout_body,
        grid=(NS // _RB,),
        in_specs=[pl.BlockSpec((_RB, ND), lambda b: (b, 0))],
        out_specs=[pl.BlockSpec((1, 1, _RB), lambda b: (b, 0, 0)),
                   pl.BlockSpec((_RB, ND), lambda b: (b, 0))],
        out_shape=[jax.ShapeDtypeStruct((NS // _RB, 1, _RB), jnp.float32),
                   jax.ShapeDtypeStruct((NS, ND), jnp.float32)],
    )(scout)


_CH = 1024
_NCH = NS // _CH
_NSP = 2048
_NCAND = 2304


def _count_below(fit, s_col):
    c = jnp.zeros((_NSP, 1), jnp.float32)
    one = jnp.ones((_NSP, _CH), jnp.float32)
    zero = jnp.zeros((_NSP, _CH), jnp.float32)
    for k in range(_NCH):
        c = c + jnp.sum(jnp.where(fit[k:k + 1, :] < s_col, one, zero),
                        axis=1, keepdims=True)
    return c


def _select_body(fit_ref, src_ref, efit_ref):
    fit = fit_ref[...].reshape(_NCH, _CH)
    ne_f = np.float32(NE)
    lo = jnp.min(fit)
    hi = jnp.max(fit) * np.float32(1.0 + 1e-5) + np.float32(1e-30)
    b_io = (lax.broadcasted_iota(jnp.int32, (_NSP, 1), 0).astype(jnp.float32)
            + np.float32(1.0))
    s1 = lo + b_io * ((hi - lo) / np.float32(_NSP))
    c1 = _count_below(fit, s1)
    lo2 = jnp.maximum(jnp.max(jnp.where(c1 < ne_f, s1, -BIG)), lo)
    hi2 = jnp.min(jnp.where(c1 >= ne_f, s1, BIG))
    s2 = lo2 + b_io * ((hi2 - lo2) / np.float32(_NSP))
    c2 = _count_below(fit, s2)
    t = jnp.minimum(jnp.min(jnp.where(c2 >= ne_f, s2, BIG)), hi2)
    tri = (lax.broadcasted_iota(jnp.int32, (_CH, _CH), 0)
           < lax.broadcasted_iota(jnp.int32, (_CH, _CH), 1))
    p_row = lax.broadcasted_iota(jnp.int32, (1, _NCAND), 1).astype(jnp.float32)
    comp_fit = jnp.zeros((1, _NCAND), jnp.float32)
    comp_idx = jnp.zeros((1, _NCAND), jnp.float32)
    matched = jnp.zeros((1, _NCAND), jnp.float32)
    offs = jnp.float32(0.0)
    zc = jnp.zeros((_CH, _CH), jnp.float32)
    zn = jnp.zeros((_CH, _NCAND), jnp.float32)
    for k in range(_NCH):
        frow = fit[k:k + 1, :]
        fcol = _row_to_col(frow)
        cand_col = fcol < t
        w = jnp.sum(jnp.where(cand_col & tri, jnp.ones_like(zc), zc),
                    axis=0, keepdims=True)
        pc_col = _row_to_col(w) + offs
        offs = offs + jnp.sum(jnp.where(cand_col, jnp.ones((_CH, 1),
                                                           jnp.float32),
                                        jnp.zeros((_CH, 1), jnp.float32)))
        m = cand_col & (pc_col == p_row)
        comp_fit = comp_fit + jnp.sum(jnp.where(m, fcol, zn), axis=0,
                                      keepdims=True)
        i_col = (lax.broadcasted_iota(jnp.int32, (_CH, 1), 0)
                 + k * _CH).astype(jnp.float32)
        comp_idx = comp_idx + jnp.sum(jnp.where(m, i_col, zn), axis=0,
                                      keepdims=True)
        matched = matched + jnp.sum(jnp.where(m, jnp.ones_like(zn), zn),
                                    axis=0, keepdims=True)
    comp_fit = comp_fit + (np.float32(1.0) - matched) * BIG
    mine_col = _row_to_col(comp_fit)
    tri_c = (lax.broadcasted_iota(jnp.int32, (_NCAND, _NCAND), 0)
             < lax.broadcasted_iota(jnp.int32, (_NCAND, _NCAND), 1))
    less = comp_fit < mine_col
    tie = (comp_fit == mine_col) & tri_c
    rank_col = jnp.sum((less | tie).astype(jnp.float32), axis=1,
                       keepdims=True)
    slot_row = lax.broadcasted_iota(jnp.int32, (1, NE), 1).astype(jnp.float32)
    idx_col = _row_to_col(comp_idx)
    fit_col = mine_col
    eq = rank_col == slot_row
    ze = jnp.zeros((_NCAND, NE), jnp.float32)
    src_ref[...] = jnp.sum(jnp.where(eq, idx_col, ze), axis=0,
                           keepdims=True).astype(jnp.int32)
    efit_ref[...] = jnp.sum(jnp.where(eq, fit_col, ze), axis=0, keepdims=True)


def _stage_bc(fit):
    return pl.pallas_call(
        _select_body,
        in_specs=[pl.BlockSpec((_NCH, 1, _CH), lambda: (0, 0, 0))],
        out_specs=[pl.BlockSpec((1, NE), lambda: (0, 0)),
                   pl.BlockSpec((1, NE), lambda: (0, 0))],
        out_shape=[jax.ShapeDtypeStruct((1, NE), jnp.int32),
                   jax.ShapeDtypeStruct((1, NE), jnp.float32)],
    )(fit)


def _sc_gather_rows(table, idx):
    b_tot = idx.shape[0]
    nw = 32
    bpw = b_tot // nw
    mesh = plsc.VectorSubcoreMesh(core_axis_name="c", subcore_axis_name="s")

    @functools.partial(
        pl.kernel, mesh=mesh,
        out_type=jax.ShapeDtypeStruct((b_tot, ND), jnp.float32),
        scratch_types=[pltpu.VMEM((bpw,), jnp.int32),
                       pltpu.VMEM((bpw, ND), jnp.float32),
                       pltpu.SemaphoreType.DMA],
    )
    def k(table_hbm, idx_hbm, out_hbm, idx_v, rows_v, sem):
        wid = lax.axis_index("s") * 2 + lax.axis_index("c")
        base = wid * bpw
        pltpu.sync_copy(idx_hbm.at[pl.ds(base, bpw)], idx_v)
        pltpu.async_copy(table_hbm.at[idx_v], rows_v, sem).wait()
        pltpu.sync_copy(rows_v, out_hbm.at[pl.ds(base, bpw)])

    return k(table, idx)


_EB = 256


def _elite_body(ebase_ref, efit_ref, epos_ref, efit2_ref):
    e = pl.program_id(0)
    fit = _row_to_col(efit_ref[...])
    d_io = lax.broadcasted_iota(jnp.int32, (_EB, ND), 1)
    pos = ebase_ref[...]
    slot = e * _EB + lax.broadcasted_iota(jnp.int32, (_EB, ND), 0)
    cnt_e = (slot * ND + d_io).astype(jnp.uint32)
    for t in range(3):
        local = pos + (_normal(_KK[t], cnt_e) * SR) * HALF
        lfit = jnp.sqrt(jnp.sum(local * local, axis=1, keepdims=True))
        imp = lfit < fit
        pos = jnp.where(imp, local, pos)
        fit = jnp.where(imp, lfit, fit)
    epos_ref[...] = pos
    efit2_ref[...] = _col_to_row(fit)


def _stage_e(ebase, efit):
    return pl.pallas_call(
        _elite_body,
        grid=(NE // _EB,),
        in_specs=[pl.BlockSpec((_EB, ND), lambda e: (e, 0)),
                  pl.BlockSpec((1, _EB), lambda e: (0, e))],
        out_specs=[pl.BlockSpec((_EB, ND), lambda e: (e, 0)),
                   pl.BlockSpec((1, _EB), lambda e: (0, e))],
        out_shape=[jax.ShapeDtypeStruct((NE, ND), jnp.float32),
                   jax.ShapeDtypeStruct((1, NE), jnp.float32)],
    )(ebase, efit)


_FB = 512


def _cat_body(efit2_ref, eidx_ref):
    b = pl.program_id(0)
    logits = -efit2_ref[...]
    s_io = b * _FB + lax.broadcasted_iota(jnp.int32, (_FB, NE), 0)
    e_io = lax.broadcasted_iota(jnp.int32, (_FB, NE), 1)
    cnt = (s_io * NE + e_io).astype(jnp.uint32)
    u = jnp.maximum(_mantissa(_tf_bits(_K3, cnt)), TINY)
    val = -jnp.log(-jnp.log(u)) + logits
    mx = jnp.max(val, axis=1, keepdims=True)
    idx = jnp.min(jnp.where(val == mx, e_io.astype(jnp.float32), BIG),
                  axis=1, keepdims=True)
    eidx_ref[...] = _col_to_row(idx).astype(jnp.int32).reshape(1, 1, _FB)


def _stage_f(efit2):
    return pl.pallas_call(
        _cat_body,
        grid=(NO // _FB,),
        in_specs=[pl.BlockSpec((1, NE), lambda b: (0, 0))],
        out_specs=pl.BlockSpec((1, 1, _FB), lambda b: (b, 0, 0)),
        out_shape=jax.ShapeDtypeStruct((NO // _FB, 1, _FB), jnp.int32),
    )(efit2)


_OB = 512


def _onlooker_body(obase_ref, efit2_ref, efinal_ref, bp_ref, bf_ref, out_ref,
                   bfit_s, brow_s):
    b = pl.program_id(0)
    obase = obase_ref[...]
    rows = b * _OB + lax.broadcasted_iota(jnp.int32, (_OB, ND), 0)
    cols = lax.broadcasted_iota(jnp.int32, (_OB, ND), 1)
    cnt = (rows * ND + cols).astype(jnp.uint32)
    local = obase + (_normal(_K4, cnt) * SR) * P3
    lfit = jnp.sqrt(jnp.sum(local * local, axis=1, keepdims=True))
    cfit = jnp.sqrt(jnp.sum(obase * obase, axis=1, keepdims=True))
    pos = jnp.where(lfit < cfit, local, obase)
    bm = jnp.min(lfit)
    r_io = lax.broadcasted_iota(jnp.int32, (_OB, 1), 0).astype(jnp.float32)
    rstar = jnp.min(jnp.where(lfit == bm, r_io, BIG))
    mask = r_io == rstar
    cand = jnp.sum(jnp.where(mask, pos, jnp.zeros_like(pos)),
                   axis=0, keepdims=True)
    bm_v = jnp.zeros((1, 128), jnp.float32) + bm

    @pl.when(b == 0)
    def _():
        bfit_s[...] = jnp.full((1, 128), BIG, jnp.float32)

    prev = bfit_s[...]
    upd = bm_v[:, 0:1] < prev[:, 0:1]
    bfit_s[...] = jnp.where(upd, bm_v, prev)
    brow_s[...] = jnp.where(upd, cand, brow_s[...])

    @pl.when(b == NO // _OB - 1)
    def _():
        efit2 = efit2_ref[...]
        em = jnp.min(efit2)
        e_io = lax.broadcasted_iota(jnp.int32, (1, NE), 1).astype(jnp.float32)
        estar = jnp.min(jnp.where(efit2 == em, e_io, BIG))
        r_io2 = lax.broadcasted_iota(jnp.int32, (NE, 1), 0).astype(jnp.float32)
        efinal = efinal_ref[...]
        erow = jnp.sum(jnp.where(r_io2 == estar, efinal,
                                 jnp.zeros_like(efinal)),
                       axis=0, keepdims=True)
        om = bfit_s[...][:, 0:1]
        em_v = jnp.zeros((1, 1), jnp.float32) + em
        ewin = em_v <= om
        brow = jnp.where(ewin, erow, brow_s[...])
        bfit = jnp.minimum(em_v, om)
        better = bfit < bf_ref[0:1, 0:1]
        outrow = jnp.where(better, brow, bp_ref[...])
        out_ref[...] = jnp.broadcast_to(outrow, out_ref.shape)


def _stage_hi(obase, efit2, efinal, bp, bf, batch):
    return pl.pallas_call(
        _onlooker_body,
        grid=(NO // _OB,),
        in_specs=[pl.BlockSpec((_OB, ND), lambda b: (b, 0)),
                  pl.BlockSpec((1, NE), lambda b: (0, 0)),
                  pl.BlockSpec((NE, ND), lambda b: (0, 0)),
                  pl.BlockSpec((1, ND), lambda b: (0, 0)),
                  pl.BlockSpec((1, 128), lambda b: (0, 0))],
        out_specs=pl.BlockSpec((batch, ND), lambda b: (0, 0)),
        out_shape=jax.ShapeDtypeStruct((batch, ND), jnp.float32),
        scratch_shapes=[pltpu.VMEM((1, 128), jnp.float32),
                        pltpu.VMEM((1, ND), jnp.float32)],
    )(obase, efit2, efinal, bp, bf)


def kernel(x, scout_positions, elite_positions, onlooker_positions,
           best_position, best_fitness):
    batch = x.shape[0]
    fit, scout_new = _stage_a(scout_positions)
    src, efit = _stage_bc(fit)
    ebase = _sc_gather_rows(scout_new, src.reshape(NE))
    efinal, efit2 = _stage_e(ebase, efit)
    eidx = _stage_f(efit2)
    obase = _sc_gather_rows(efinal, eidx.reshape(NO))
    bp = best_position.reshape(1, ND)
    bf = jnp.broadcast_to(best_fitness.reshape(1, 1), (1, 128))
    return _stage_hi(obase, efit2, efinal, bp, bf, batch)


# speedup vs baseline: 1.6018x; 1.0183x over previous
import functools

import jax
import jax.numpy as jnp
import numpy as np
from jax import lax
from jax.experimental import pallas as pl
from jax.experimental.pallas import tpu as pltpu
from jax.experimental.pallas import tpu_sc as plsc

NS = 16384
NE = 2048
NO = 4096
ND = 512

SR = np.float32(0.1)
HALF = np.float32(0.5)
P3 = np.float32(0.3)
LO = np.float32(np.nextafter(np.float32(-1.0), np.float32(0.0)))
SQRT2 = np.float32(np.sqrt(2.0))
TINY = np.float32(np.finfo(np.float32).tiny)
BIG = np.float32(1e9)

M32 = 0xFFFFFFFF


def _tf_host(k, c):
    def rotl(x, d):
        return ((x << d) | (x >> (32 - d))) & M32
    ks = [k[0], k[1], 0x1BD11BDA ^ k[0] ^ k[1]]
    x0 = (c[0] + ks[0]) & M32
    x1 = (c[1] + ks[1]) & M32
    rots = [[13, 15, 26, 6], [17, 29, 16, 24]]
    for i in range(5):
        for r in rots[i % 2]:
            x0 = (x0 + x1) & M32
            x1 = rotl(x1, r)
            x1 = x0 ^ x1
        x0 = (x0 + ks[(i + 1) % 3]) & M32
        x1 = (x1 + ks[(i + 2) % 3] + i + 1) & M32
    return x0, x1


_KEY = (0, 42)
_K1 = _tf_host(_KEY, (0, 0))
_K2 = _tf_host(_KEY, (0, 1))
_K3 = _tf_host(_KEY, (0, 2))
_K4 = _tf_host(_KEY, (0, 3))
_KK = tuple(_tf_host(_K2, (0, t)) for t in range(3))


def _tf_bits(key, cnt):
    ks = [jnp.uint32(key[0]), jnp.uint32(key[1]),
          jnp.uint32(0x1BD11BDA ^ key[0] ^ key[1])]
    x0 = jnp.full(cnt.shape, ks[0], jnp.uint32)
    x1 = cnt + ks[1]
    rots = ((13, 15, 26, 6), (17, 29, 16, 24))
    for g in range(5):
        for r in rots[g % 2]:
            x0 = x0 + x1
            x1 = (x1 << jnp.uint32(r)) | (x1 >> jnp.uint32(32 - r))
            x1 = x0 ^ x1
        x0 = x0 + ks[(g + 1) % 3]
        x1 = x1 + ks[(g + 2) % 3] + jnp.uint32(g + 1)
    return x0 ^ x1


def _mantissa(bits):
    fb = (bits >> jnp.uint32(9)) | jnp.uint32(0x3F800000)
    return lax.bitcast_convert_type(fb, jnp.float32) - jnp.float32(1.0)


_ERF_A = [2.81022636e-08, 3.43273939e-07, -3.5233877e-06, -4.39150654e-06,
          0.00021858087, -0.00125372503, -0.00417768164, 0.246640727,
          1.50140941]
_ERF_B = [-0.000200214257, 0.000100950558, 0.00134934322, -0.00367342844,
          0.00573950773, -0.0076224613, 0.00943887047, 1.00167406,
          2.83297682]


def _erf_inv(x):
    w = -jnp.log1p(x * -x)
    wl = w - np.float32(2.5)
    wg = jnp.sqrt(w) - np.float32(3.0)
    pa = jnp.full_like(x, np.float32(_ERF_A[0]))
    pb = jnp.full_like(x, np.float32(_ERF_B[0]))
    for i in range(1, 9):
        pa = np.float32(_ERF_A[i]) + pa * wl
        pb = np.float32(_ERF_B[i]) + pb * wg
    return jnp.where(w < np.float32(5.0), pa, pb) * x


def _normal(key, cnt):
    m = _mantissa(_tf_bits(key, cnt))
    u = jnp.maximum(LO, m * jnp.float32(2.0) + LO)
    return SQRT2 * _erf_inv(u)


def _col_to_row(col):
    n = col.shape[0]
    sel = (lax.broadcasted_iota(jnp.int32, (n, n), 0)
           == lax.broadcasted_iota(jnp.int32, (n, n), 1))
    return jnp.sum(jnp.where(sel, col, jnp.zeros_like(col)), axis=0,
                   keepdims=True)


def _row_to_col(row):
    n = row.shape[1]
    sel = (lax.broadcasted_iota(jnp.int32, (n, n), 0)
           == lax.broadcasted_iota(jnp.int32, (n, n), 1))
    return jnp.sum(jnp.where(sel, row, jnp.zeros_like(row)), axis=1,
                   keepdims=True)


_RB = 1024


def _scout_body(scout_ref, fit_ref, pos_ref):
    b = pl.program_id(0)
    pos = scout_ref[...]
    rows = lax.broadcasted_iota(jnp.int32, (_RB, ND), 0) + b * _RB
    cols = lax.broadcasted_iota(jnp.int32, (_RB, ND), 1)
    cnt = (rows * ND + cols).astype(jnp.uint32)
    pos = pos + _normal(_K1, cnt) * SR
    pos_ref[...] = pos
    fit = jnp.sqrt(jnp.sum(pos * pos, axis=1, keepdims=True))
    fit_ref[...] = _col_to_row(fit).reshape(1, 1, _RB)


def _stage_a(scout):
    return pl.pallas_call(
        _scout_body,
        grid=(NS // _RB,),
        in_specs=[pl.BlockSpec((_RB, ND), lambda b: (b, 0))],
        out_specs=[pl.BlockSpec((1, 1, _RB), lambda b: (b, 0, 0)),
                   pl.BlockSpec((_RB, ND), lambda b: (b, 0))],
        out_shape=[jax.ShapeDtypeStruct((NS // _RB, 1, _RB), jnp.float32),
                   jax.ShapeDtypeStruct((NS, ND), jnp.float32)],
    )(scout)


_CH = 1024
_NCH = NS // _CH
_NSP = 2048
_NCAND = 2304


def _count_below(fit, s_col):
    c = jnp.zeros((_NSP, 1), jnp.float32)
    one = jnp.ones((_NSP, _CH), jnp.float32)
    zero = jnp.zeros((_NSP, _CH), jnp.float32)
    for k in range(_NCH):
        c = c + jnp.sum(jnp.where(fit[k:k + 1, :] < s_col, one, zero),
                        axis=1, keepdims=True)
    return c


def _select_body(fit_ref, src_ref, efit_ref):
    fit = fit_ref[...].reshape(_NCH, _CH)
    ne_f = np.float32(NE)
    lo = jnp.min(fit)
    hi = jnp.max(fit) * np.float32(1.0 + 1e-5) + np.float32(1e-30)
    b_io = (lax.broadcasted_iota(jnp.int32, (_NSP, 1), 0).astype(jnp.float32)
            + np.float32(1.0))
    s1 = lo + b_io * ((hi - lo) / np.float32(_NSP))
    c1 = _count_below(fit, s1)
    lo2 = jnp.maximum(jnp.max(jnp.where(c1 < ne_f, s1, -BIG)), lo)
    hi2 = jnp.min(jnp.where(c1 >= ne_f, s1, BIG))
    s2 = lo2 + b_io * ((hi2 - lo2) / np.float32(_NSP))
    c2 = _count_below(fit, s2)
    t = jnp.minimum(jnp.min(jnp.where(c2 >= ne_f, s2, BIG)), hi2)
    tri = (lax.broadcasted_iota(jnp.int32, (_CH, _CH), 0)
           < lax.broadcasted_iota(jnp.int32, (_CH, _CH), 1))
    p_row = lax.broadcasted_iota(jnp.int32, (1, _NCAND), 1).astype(jnp.float32)
    comp_fit = jnp.zeros((1, _NCAND), jnp.float32)
    comp_idx = jnp.zeros((1, _NCAND), jnp.float32)
    matched = jnp.zeros((1, _NCAND), jnp.float32)
    offs = jnp.float32(0.0)
    zc = jnp.zeros((_CH, _CH), jnp.float32)
    zn = jnp.zeros((_CH, _NCAND), jnp.float32)
    for k in range(_NCH):
        frow = fit[k:k + 1, :]
        fcol = _row_to_col(frow)
        cand_col = fcol < t
        w = jnp.sum(jnp.where(cand_col & tri, jnp.ones_like(zc), zc),
                    axis=0, keepdims=True)
        pc_col = _row_to_col(w) + offs
        offs = offs + jnp.sum(jnp.where(cand_col, jnp.ones((_CH, 1),
                                                           jnp.float32),
                                        jnp.zeros((_CH, 1), jnp.float32)))
        m = cand_col & (pc_col == p_row)
        comp_fit = comp_fit + jnp.sum(jnp.where(m, fcol, zn), axis=0,
                                      keepdims=True)
        i_col = (lax.broadcasted_iota(jnp.int32, (_CH, 1), 0)
                 + k * _CH).astype(jnp.float32)
        comp_idx = comp_idx + jnp.sum(jnp.where(m, i_col, zn), axis=0,
                                      keepdims=True)
        matched = matched + jnp.sum(jnp.where(m, jnp.ones_like(zn), zn),
                                    axis=0, keepdims=True)
    comp_fit = comp_fit + (np.float32(1.0) - matched) * BIG
    mine_col = _row_to_col(comp_fit)
    tri_c = (lax.broadcasted_iota(jnp.int32, (_NCAND, _NCAND), 0)
             < lax.broadcasted_iota(jnp.int32, (_NCAND, _NCAND), 1))
    less = comp_fit < mine_col
    tie = (comp_fit == mine_col) & tri_c
    rank_col = jnp.sum((less | tie).astype(jnp.float32), axis=1,
                       keepdims=True)
    slot_row = lax.broadcasted_iota(jnp.int32, (1, NE), 1).astype(jnp.float32)
    idx_col = _row_to_col(comp_idx)
    fit_col = mine_col
    eq = rank_col == slot_row
    ze = jnp.zeros((_NCAND, NE), jnp.float32)
    src_ref[...] = jnp.sum(jnp.where(eq, idx_col, ze), axis=0,
                           keepdims=True).astype(jnp.int32)
    efit_ref[...] = jnp.sum(jnp.where(eq, fit_col, ze), axis=0, keepdims=True)


def _stage_bc(fit):
    return pl.pallas_call(
        _select_body,
        in_specs=[pl.BlockSpec((_NCH, 1, _CH), lambda: (0, 0, 0))],
        out_specs=[pl.BlockSpec((1, NE), lambda: (0, 0)),
                   pl.BlockSpec((1, NE), lambda: (0, 0))],
        out_shape=[jax.ShapeDtypeStruct((1, NE), jnp.int32),
                   jax.ShapeDtypeStruct((1, NE), jnp.float32)],
    )(fit)


def _sc_gather_rows(table, idx):
    b_tot = idx.shape[0]
    nw = 32
    bpw = b_tot // nw
    mesh = plsc.VectorSubcoreMesh(core_axis_name="c", subcore_axis_name="s")

    @functools.partial(
        pl.kernel, mesh=mesh,
        out_type=jax.ShapeDtypeStruct((b_tot, ND), jnp.float32),
        scratch_types=[pltpu.VMEM((bpw,), jnp.int32),
                       pltpu.VMEM((bpw, ND), jnp.float32),
                       pltpu.SemaphoreType.DMA],
    )
    def k(table_hbm, idx_hbm, out_hbm, idx_v, rows_v, sem):
        wid = lax.axis_index("s") * 2 + lax.axis_index("c")
        base = wid * bpw
        pltpu.sync_copy(idx_hbm.at[pl.ds(base, bpw)], idx_v)
        pltpu.async_copy(table_hbm.at[idx_v], rows_v, sem).wait()
        pltpu.sync_copy(rows_v, out_hbm.at[pl.ds(base, bpw)])

    return k(table, idx)


_EB = 256


def _elite_body(ebase_ref, efit_ref, epos_ref, efit2_ref):
    e = pl.program_id(0)
    fit = _row_to_col(efit_ref[...])
    d_io = lax.broadcasted_iota(jnp.int32, (_EB, ND), 1)
    pos = ebase_ref[...]
    slot = e * _EB + lax.broadcasted_iota(jnp.int32, (_EB, ND), 0)
    cnt_e = (slot * ND + d_io).astype(jnp.uint32)
    for t in range(3):
        local = pos + (_normal(_KK[t], cnt_e) * SR) * HALF
        lfit = jnp.sqrt(jnp.sum(local * local, axis=1, keepdims=True))
        imp = lfit < fit
        pos = jnp.where(imp, local, pos)
        fit = jnp.where(imp, lfit, fit)
    epos_ref[...] = pos
    efit2_ref[...] = _col_to_row(fit)


def _stage_e(ebase, efit):
    return pl.pallas_call(
        _elite_body,
        grid=(NE // _EB,),
        in_specs=[pl.BlockSpec((_EB, ND), lambda e: (e, 0)),
                  pl.BlockSpec((1, _EB), lambda e: (0, e))],
        out_specs=[pl.BlockSpec((_EB, ND), lambda e: (e, 0)),
                   pl.BlockSpec((1, _EB), lambda e: (0, e))],
        out_shape=[jax.ShapeDtypeStruct((NE, ND), jnp.float32),
                   jax.ShapeDtypeStruct((1, NE), jnp.float32)],
    )(ebase, efit)


_FB = 512
_SC_R0 = 2048


def _gumbel_argmax(bits, logits, row0, eidx_ref):
    e_io = lax.broadcasted_iota(jnp.int32, (_FB, NE), 1)
    u = jnp.maximum(_mantissa(bits), TINY)
    val = -jnp.log(-jnp.log(u)) + logits
    mx = jnp.max(val, axis=1, keepdims=True)
    idx = jnp.min(jnp.where(val == mx, e_io.astype(jnp.float32), BIG),
                  axis=1, keepdims=True)
    eidx_ref[...] = _col_to_row(idx).astype(jnp.int32).reshape(1, 1, _FB)


def _cat_tc_body(efit2_ref, eidx_ref):
    b = pl.program_id(0)
    s_io = b * _FB + lax.broadcasted_iota(jnp.int32, (_FB, NE), 0)
    e_io = lax.broadcasted_iota(jnp.int32, (_FB, NE), 1)
    cnt = (s_io * NE + e_io).astype(jnp.uint32)
    _gumbel_argmax(_tf_bits(_K3, cnt), -efit2_ref[...], b * _FB, eidx_ref)


def _cat_sc_body(efit2_ref, bits_ref, eidx_ref):
    b = pl.program_id(0)
    bits = lax.bitcast_convert_type(bits_ref[...], jnp.uint32)
    _gumbel_argmax(bits, -efit2_ref[...], _SC_R0 + b * _FB, eidx_ref)


def _stage_f(efit2, sc_bits):
    eidx1 = pl.pallas_call(
        _cat_tc_body,
        grid=(_SC_R0 // _FB,),
        in_specs=[pl.BlockSpec((1, NE), lambda b: (0, 0))],
        out_specs=pl.BlockSpec((1, 1, _FB), lambda b: (b, 0, 0)),
        out_shape=jax.ShapeDtypeStruct((_SC_R0 // _FB, 1, _FB), jnp.int32),
    )(efit2)
    nr = NO - _SC_R0
    eidx2 = pl.pallas_call(
        _cat_sc_body,
        grid=(nr // _FB,),
        in_specs=[pl.BlockSpec((1, NE), lambda b: (0, 0)),
                  pl.BlockSpec((_FB, NE), lambda b: (b, 0))],
        out_specs=pl.BlockSpec((1, 1, _FB), lambda b: (b, 0, 0)),
        out_shape=jax.ShapeDtypeStruct((nr // _FB, 1, _FB), jnp.int32),
    )(efit2, sc_bits)
    return jnp.concatenate([eidx1.reshape(_SC_R0), eidx2.reshape(nr)])


def _sc_gumbel_bits():
    nr = NO - _SC_R0
    nw = 32
    rpw = nr // nw
    wpw = rpw * NE
    chw = 16384
    nch = wpw // chw
    mesh = plsc.VectorSubcoreMesh(core_axis_name="c", subcore_axis_name="s")

    @functools.partial(
        pl.kernel, mesh=mesh,
        out_type=jax.ShapeDtypeStruct((nr * NE,), jnp.int32),
        scratch_types=[pltpu.VMEM((chw,), jnp.int32)],
    )
    def k(out_hbm, buf):
        wid = lax.axis_index("s") * 2 + lax.axis_index("c")
        cbase = (_SC_R0 + wid * rpw) * NE
        obase = wid * wpw
        io16 = lax.iota(jnp.int32, 16)
        for ch in range(nch):
            def body(v, carry, ch=ch):
                c0 = cbase + ch * chw + v * 16
                cnt = (io16 + c0).astype(jnp.uint32)
                bits = _tf_bits(_K3, cnt)
                buf[pl.ds(v * 16, 16)] = lax.bitcast_convert_type(bits,
                                                                  jnp.int32)
                return carry
            lax.fori_loop(0, chw // 16, body, 0)
            pltpu.sync_copy(buf, out_hbm.at[pl.ds(obase + ch * chw, chw)])

    return k().reshape(nr, NE)


_OB = 512


def _onlooker_body(obase_ref, efit2_ref, efinal_ref, bp_ref, bf_ref, out_ref,
                   bfit_s, brow_s):
    b = pl.program_id(0)
    obase = obase_ref[...]
    rows = b * _OB + lax.broadcasted_iota(jnp.int32, (_OB, ND), 0)
    cols = lax.broadcasted_iota(jnp.int32, (_OB, ND), 1)
    cnt = (rows * ND + cols).astype(jnp.uint32)
    local = obase + (_normal(_K4, cnt) * SR) * P3
    lfit = jnp.sqrt(jnp.sum(local * local, axis=1, keepdims=True))
    cfit = jnp.sqrt(jnp.sum(obase * obase, axis=1, keepdims=True))
    pos = jnp.where(lfit < cfit, local, obase)
    bm = jnp.min(lfit)
    r_io = lax.broadcasted_iota(jnp.int32, (_OB, 1), 0).astype(jnp.float32)
    rstar = jnp.min(jnp.where(lfit == bm, r_io, BIG))
    mask = r_io == rstar
    cand = jnp.sum(jnp.where(mask, pos, jnp.zeros_like(pos)),
                   axis=0, keepdims=True)
    bm_v = jnp.zeros((1, 128), jnp.float32) + bm

    @pl.when(b == 0)
    def _():
        bfit_s[...] = jnp.full((1, 128), BIG, jnp.float32)

    prev = bfit_s[...]
    upd = bm_v[:, 0:1] < prev[:, 0:1]
    bfit_s[...] = jnp.where(upd, bm_v, prev)
    brow_s[...] = jnp.where(upd, cand, brow_s[...])

    @pl.when(b == NO // _OB - 1)
    def _():
        efit2 = efit2_ref[...]
        em = jnp.min(efit2)
        e_io = lax.broadcasted_iota(jnp.int32, (1, NE), 1).astype(jnp.float32)
        estar = jnp.min(jnp.where(efit2 == em, e_io, BIG))
        r_io2 = lax.broadcasted_iota(jnp.int32, (NE, 1), 0).astype(jnp.float32)
        efinal = efinal_ref[...]
        erow = jnp.sum(jnp.where(r_io2 == estar, efinal,
                                 jnp.zeros_like(efinal)),
                       axis=0, keepdims=True)
        om = bfit_s[...][:, 0:1]
        em_v = jnp.zeros((1, 1), jnp.float32) + em
        ewin = em_v <= om
        brow = jnp.where(ewin, erow, brow_s[...])
        bfit = jnp.minimum(em_v, om)
        better = bfit < bf_ref[0:1, 0:1]
        outrow = jnp.where(better, brow, bp_ref[...])
        out_ref[...] = jnp.broadcast_to(outrow, out_ref.shape)


def _stage_hi(obase, efit2, efinal, bp, bf, batch):
    return pl.pallas_call(
        _onlooker_body,
        grid=(NO // _OB,),
        in_specs=[pl.BlockSpec((_OB, ND), lambda b: (b, 0)),
                  pl.BlockSpec((1, NE), lambda b: (0, 0)),
                  pl.BlockSpec((NE, ND), lambda b: (0, 0)),
                  pl.BlockSpec((1, ND), lambda b: (0, 0)),
                  pl.BlockSpec((1, 128), lambda b: (0, 0))],
        out_specs=pl.BlockSpec((batch, ND), lambda b: (0, 0)),
        out_shape=jax.ShapeDtypeStruct((batch, ND), jnp.float32),
        scratch_shapes=[pltpu.VMEM((1, 128), jnp.float32),
                        pltpu.VMEM((1, ND), jnp.float32)],
    )(obase, efit2, efinal, bp, bf)


def kernel(x, scout_positions, elite_positions, onlooker_positions,
           best_position, best_fitness):
    batch = x.shape[0]
    sc_bits = _sc_gumbel_bits()
    fit, scout_new = _stage_a(scout_positions)
    src, efit = _stage_bc(fit)
    ebase = _sc_gather_rows(scout_new, src.reshape(NE))
    efinal, efit2 = _stage_e(ebase, efit)
    eidx = _stage_f(efit2, sc_bits)
    obase = _sc_gather_rows(efinal, eidx)
    bp = best_position.reshape(1, ND)
    bf = jnp.broadcast_to(best_fitness.reshape(1, 1), (1, 128))
    return _stage_hi(obase, efit2, efinal, bp, bf, batch)
